# Initial kernel scaffold; baseline (speedup 1.0000x reference)
import jax
import jax.numpy as jnp
from jax.experimental import pallas as pl


def kernel(edges, coor, W1, b1, W2, b2, W4, b4):
    raise NotImplementedError("write your pallas kernel here")


# speedup vs baseline: 26.5483x; 26.5483x over previous
import jax
import jax.numpy as jnp
from jax import lax
from jax.experimental import pallas as pl
from jax.experimental.pallas import tpu as pltpu
from jax.experimental.pallas import tpu_sc as plsc

NSEG = 100000
E_TOTAL = 6400000
NC, NS, L = 2, 16, 16
NW = NC * NS
B = 128
K = 16
T = K * B
UNITS = E_TOTAL // B
BASE_UNITS = UNITS // NW
EXTRA = UNITS - BASE_UNITS * NW
FULL_TILES = BASE_UNITS // K
SEG_PAD = 100352
STRIPE4 = SEG_PAD * 4 // NS
SEG_W = 3136
SEG_LAST = NSEG - 31 * SEG_W


def _accum_body(coor_hbm, edges_hbm, wvec_hbm, zeros_hbm, part_hbm,
                acc, cbuf, ibuf, vbuf, wibuf, wbuf, sem):
    c = lax.axis_index("c")
    s = lax.axis_index("s")
    w = s * NC + c

    pltpu.sync_copy(zeros_hbm.at[pl.ds(s * STRIPE4, STRIPE4)],
                    acc.at[pl.ds(s * STRIPE4, STRIPE4)])
    pltpu.sync_copy(wvec_hbm, wbuf)
    plsc.subcore_barrier()

    iota = lax.iota(jnp.int32, L)
    w00 = wbuf[0, :]
    w01 = wbuf[1, :]
    w02 = wbuf[2, :]
    w10 = wbuf[3, :]
    w11 = wbuf[4, :]
    w12 = wbuf[5, :]
    w20 = wbuf[6, :]
    w21 = wbuf[7, :]
    w22 = wbuf[8, :]
    wb0 = wbuf[9, :]
    wb1 = wbuf[10, :]
    wb2 = wbuf[11, :]
    fones = w00 * 0.0 + 1.0

    def _ones(r, carry):
        vbuf[pl.ds(3 * T + r * L, L)] = fones
        return carry

    lax.fori_loop(0, T // L, _ones, 0)

    unit0 = w * BASE_UNITS + jnp.minimum(w, EXTRA)
    n_units = BASE_UNITS + (w < EXTRA).astype(jnp.int32)

    def process_tile(start_unit, k):
        row0 = start_unit * B
        descs = [
            pltpu.async_copy(coor_hbm.at[pl.ds(row0, k * B)],
                             cbuf.at[pl.ds(0, k * B)], sem),
            pltpu.async_copy(coor_hbm.at[pl.ds(E_TOTAL + row0, k * B)],
                             cbuf.at[pl.ds(T, k * B)], sem),
            pltpu.async_copy(coor_hbm.at[pl.ds(2 * E_TOTAL + row0, k * B)],
                             cbuf.at[pl.ds(2 * T, k * B)], sem),
            pltpu.async_copy(edges_hbm.at[pl.ds(row0, k * B)],
                             ibuf.at[pl.ds(0, k * B)], sem),
        ]
        for d in descs:
            d.wait()

        def grp(m0, carry):
            for u in range(4):
                m = m0 * 4 + u
                r = m * L
                x0 = cbuf[pl.ds(r, L)]
                x1 = cbuf[pl.ds(T + r, L)]
                x2 = cbuf[pl.ds(2 * T + r, L)]
                vbuf[pl.ds(r, L)] = x0 * w00 + x1 * w01 + x2 * w02 + wb0
                vbuf[pl.ds(T + r, L)] = x0 * w10 + x1 * w11 + x2 * w12 + wb1
                vbuf[pl.ds(2 * T + r, L)] = (
                    x0 * w20 + x1 * w21 + x2 * w22 + wb2)
                ids = ibuf[pl.ds(r, L)]
                jb = m // 8
                jo = (m % 8) * L
                wibuf[jb, pl.ds(jo, L)] = ids
                wibuf[16 + jb, pl.ds(jo, L)] = ids + SEG_PAD
                wibuf[32 + jb, pl.ds(jo, L)] = ids + 2 * SEG_PAD
                wibuf[48 + jb, pl.ds(jo, L)] = ids + 3 * SEG_PAD
            return carry

        lax.fori_loop(0, k * 2, grp, 0)
        for cc in range(4):
            for b in range(k):
                pltpu.sync_copy(vbuf.at[pl.ds(cc * T + b * B, B)],
                                acc.at[wibuf.at[cc * 16 + b]], add=True)

    def full_tile(t, carry):
        process_tile(unit0 + t * K, K)
        return carry

    lax.fori_loop(0, FULL_TILES, full_tile, 0)

    def rem_unit(u, carry):
        process_tile(unit0 + FULL_TILES * K + u, 1)
        return carry

    lax.fori_loop(0, n_units - FULL_TILES * K, rem_unit, 0)

    plsc.subcore_barrier()
    pltpu.sync_copy(acc.at[pl.ds(s * STRIPE4, STRIPE4)],
                    part_hbm.at[pl.ds(c * (SEG_PAD * 4) + s * STRIPE4,
                                      STRIPE4)])


def _final_body(part_hbm, out_hbm, pbuf, obuf, sem):
    c = lax.axis_index("c")
    s = lax.axis_index("s")
    w = s * NC + c
    seg0 = w * SEG_W

    def run(nseg):
        descs = []
        for sc in range(NC):
            for cc in range(4):
                descs.append(pltpu.async_copy(
                    part_hbm.at[pl.ds(
                        sc * (SEG_PAD * 4) + cc * SEG_PAD + seg0, nseg)],
                    pbuf.at[pl.ds((sc * 4 + cc) * SEG_W, nseg)], sem))
        for d in descs:
            d.wait()

        def grp(g, carry):
            r = g * L
            a0 = pbuf[pl.ds(r, L)] + pbuf[pl.ds(4 * SEG_W + r, L)]
            a1 = (pbuf[pl.ds(SEG_W + r, L)]
                  + pbuf[pl.ds(5 * SEG_W + r, L)])
            a2 = (pbuf[pl.ds(2 * SEG_W + r, L)]
                  + pbuf[pl.ds(6 * SEG_W + r, L)])
            cnt = (pbuf[pl.ds(3 * SEG_W + r, L)]
                   + pbuf[pl.ds(7 * SEG_W + r, L)])
            rec = 1.0 / jnp.maximum(cnt, 1.0)
            obuf[pl.ds(r, L)] = a0 * rec
            obuf[pl.ds(SEG_W + r, L)] = a1 * rec
            obuf[pl.ds(2 * SEG_W + r, L)] = a2 * rec
            return carry

        lax.fori_loop(0, nseg // L, grp, 0)
        for cc in range(3):
            pltpu.sync_copy(
                obuf.at[pl.ds(cc * SEG_W, nseg)],
                out_hbm.at[pl.ds(cc * NSEG + seg0, nseg)])

    @pl.when(w < NW - 1)
    def _():
        run(SEG_W)

    @pl.when(w == NW - 1)
    def _():
        run(SEG_LAST)


def _run_accum(coor_flat, edges_flat, wvec, zeros):
    accum = pl.kernel(
        _accum_body,
        out_type=jax.ShapeDtypeStruct((NC * SEG_PAD * 4,), jnp.float32),
        mesh=plsc.VectorSubcoreMesh(core_axis_name="c",
                                    subcore_axis_name="s"),
        compiler_params=pltpu.CompilerParams(needs_layout_passes=False),
        scratch_types=[
            pltpu.VMEM_SHARED((SEG_PAD * 4,), jnp.float32),
            pltpu.VMEM((T * 3,), jnp.float32),
            pltpu.VMEM((T,), jnp.int32),
            pltpu.VMEM((T * 4,), jnp.float32),
            pltpu.VMEM((K * 4, B), jnp.int32),
            pltpu.VMEM((12, L), jnp.float32),
            pltpu.SemaphoreType.DMA,
        ],
    )
    return accum(coor_flat, edges_flat, wvec, zeros)


def _run_final(part):
    final = pl.kernel(
        _final_body,
        out_type=jax.ShapeDtypeStruct((3 * NSEG,), jnp.float32),
        mesh=plsc.VectorSubcoreMesh(core_axis_name="c",
                                    subcore_axis_name="s"),
        compiler_params=pltpu.CompilerParams(needs_layout_passes=False),
        scratch_types=[
            pltpu.VMEM((8 * SEG_W,), jnp.float32),
            pltpu.VMEM((3 * SEG_W,), jnp.float32),
            pltpu.SemaphoreType.DMA,
        ],
    )
    return final(part)


@jax.jit
def kernel(edges, coor, W1, b1, W2, b2, W4, b4):
    Wc = W4 @ W2 @ W1
    bc = W4 @ (W2 @ b1 + b2) + b4
    wvec = jnp.broadcast_to(
        jnp.concatenate([Wc.reshape(9), bc])[:, None], (12, L))
    coor_flat = jnp.transpose(coor, (0, 2, 1)).reshape(3 * E_TOTAL)
    edges_flat = edges.reshape(E_TOTAL)
    zeros = jnp.zeros((SEG_PAD * 4,), jnp.float32)
    part = _run_accum(coor_flat, edges_flat, wvec, zeros)
    out3 = _run_final(part)
    return jnp.transpose(out3.reshape(3, NSEG))[None]


# speedup vs baseline: 33.6781x; 1.2686x over previous
import jax
import jax.numpy as jnp
from jax import lax
from jax.experimental import pallas as pl
from jax.experimental.pallas import tpu as pltpu
from jax.experimental.pallas import tpu_sc as plsc

NSEG = 100000
E_TOTAL = 6400000
NC, NS, L = 2, 16, 16
NW = NC * NS
B = 128
K = 16
T = K * B
UNITS = E_TOTAL // B
BASE_UNITS = UNITS // NW
EXTRA = UNITS - BASE_UNITS * NW
FULL_TILES = BASE_UNITS // K
SEG_PAD = 100352
STRIPE4 = SEG_PAD * 4 // NS
SEG_W = 3136
SEG_LAST = NSEG - 31 * SEG_W


def _accum_body(coor_hbm, edges_hbm, wvec_hbm, zeros_hbm, part_hbm,
                acc0, acc1, acc2, acc3,
                cbuf0, ibuf0, vbuf0, wibuf0,
                cbuf1, ibuf1, vbuf1, wibuf1,
                wbuf, sem, sem0, sem1):
    c = lax.axis_index("c")
    s = lax.axis_index("s")
    w = s * NC + c
    accs = (acc0, acc1, acc2, acc3)

    stripe = SEG_PAD // NS
    for a in accs:
        pltpu.sync_copy(zeros_hbm.at[pl.ds(s * stripe, stripe)],
                        a.at[pl.ds(s * stripe, stripe)])
    pltpu.sync_copy(wvec_hbm, wbuf)
    plsc.subcore_barrier()

    w00 = wbuf[0, :]
    w01 = wbuf[1, :]
    w02 = wbuf[2, :]
    w10 = wbuf[3, :]
    w11 = wbuf[4, :]
    w12 = wbuf[5, :]
    w20 = wbuf[6, :]
    w21 = wbuf[7, :]
    w22 = wbuf[8, :]
    wb0 = wbuf[9, :]
    wb1 = wbuf[10, :]
    wb2 = wbuf[11, :]
    fones = w00 * 0.0 + 1.0

    def _ones(r, carry):
        vbuf0[pl.ds(3 * T + r * L, L)] = fones
        vbuf1[pl.ds(3 * T + r * L, L)] = fones
        return carry

    lax.fori_loop(0, T // L, _ones, 0)

    unit0 = w * BASE_UNITS + jnp.minimum(w, EXTRA)
    n_units = BASE_UNITS + (w < EXTRA).astype(jnp.int32)

    def load_tile(start_unit, k, cbuf, ibuf):
        row0 = start_unit * B
        descs = [
            pltpu.async_copy(coor_hbm.at[pl.ds(row0, k * B)],
                             cbuf.at[pl.ds(0, k * B)], sem),
            pltpu.async_copy(coor_hbm.at[pl.ds(E_TOTAL + row0, k * B)],
                             cbuf.at[pl.ds(T, k * B)], sem),
            pltpu.async_copy(coor_hbm.at[pl.ds(2 * E_TOTAL + row0, k * B)],
                             cbuf.at[pl.ds(2 * T, k * B)], sem),
            pltpu.async_copy(edges_hbm.at[pl.ds(row0, k * B)],
                             ibuf.at[pl.ds(0, k * B)], sem),
        ]
        for d in descs:
            d.wait()

    def compute_tile(k, cbuf, ibuf, vbuf, wibuf):
        def grp(m0, carry):
            for u in range(4):
                m = m0 * 4 + u
                r = m * L
                x0 = cbuf[pl.ds(r, L)]
                x1 = cbuf[pl.ds(T + r, L)]
                x2 = cbuf[pl.ds(2 * T + r, L)]
                vbuf[pl.ds(r, L)] = x0 * w00 + x1 * w01 + x2 * w02 + wb0
                vbuf[pl.ds(T + r, L)] = x0 * w10 + x1 * w11 + x2 * w12 + wb1
                vbuf[pl.ds(2 * T + r, L)] = (
                    x0 * w20 + x1 * w21 + x2 * w22 + wb2)
                wibuf[m // 8, pl.ds((m % 8) * L, L)] = ibuf[pl.ds(r, L)]
            return carry

        lax.fori_loop(0, k * 2, grp, 0)

    def stream_tile(k, vbuf, wibuf, ssem, start):
        for b in range(k):
            for cc in range(4):
                d = pltpu.make_async_copy(
                    vbuf.at[pl.ds(cc * T + b * B, B)],
                    accs[cc].at[wibuf.at[b]], ssem)
                if start:
                    d.start(add=True)
                else:
                    d.wait()

    bufs = ((cbuf0, ibuf0, vbuf0, wibuf0, sem0),
            (cbuf1, ibuf1, vbuf1, wibuf1, sem1))

    def pair(t, carry):
        for p in range(2):
            cbuf, ibuf, vbuf, wibuf, ssem = bufs[p]

            @pl.when(t > 0)
            def _():
                stream_tile(K, vbuf, wibuf, ssem, start=False)

            load_tile(unit0 + (t * 2 + p) * K, K, cbuf, ibuf)
            compute_tile(K, cbuf, ibuf, vbuf, wibuf)
            stream_tile(K, vbuf, wibuf, ssem, start=True)
        return carry

    npairs = FULL_TILES // 2
    lax.fori_loop(0, npairs, pair, 0)
    stream_tile(K, vbuf0, wibuf0, sem0, start=False)
    stream_tile(K, vbuf1, wibuf1, sem1, start=False)

    def tail_tile(start_unit, k):
        load_tile(start_unit, k, cbuf0, ibuf0)
        compute_tile(k, cbuf0, ibuf0, vbuf0, wibuf0)
        stream_tile(k, vbuf0, wibuf0, sem0, start=True)
        stream_tile(k, vbuf0, wibuf0, sem0, start=False)

    tail_tile(unit0 + npairs * 2 * K, K)

    def rem_unit(u, carry):
        tail_tile(unit0 + FULL_TILES * K + u, 1)
        return carry

    lax.fori_loop(0, n_units - FULL_TILES * K, rem_unit, 0)

    plsc.subcore_barrier()
    for cc in range(4):
        pltpu.sync_copy(
            accs[cc].at[pl.ds(s * stripe, stripe)],
            part_hbm.at[pl.ds(c * (SEG_PAD * 4) + cc * SEG_PAD + s * stripe,
                              stripe)])


def _final_body(part_hbm, out_hbm, pbuf, obuf, sem):
    c = lax.axis_index("c")
    s = lax.axis_index("s")
    w = s * NC + c
    seg0 = w * SEG_W

    def run(nseg):
        descs = []
        for sc in range(NC):
            for cc in range(4):
                descs.append(pltpu.async_copy(
                    part_hbm.at[pl.ds(
                        sc * (SEG_PAD * 4) + cc * SEG_PAD + seg0, nseg)],
                    pbuf.at[pl.ds((sc * 4 + cc) * SEG_W, nseg)], sem))
        for d in descs:
            d.wait()

        def grp(g, carry):
            r = g * L
            a0 = pbuf[pl.ds(r, L)] + pbuf[pl.ds(4 * SEG_W + r, L)]
            a1 = (pbuf[pl.ds(SEG_W + r, L)]
                  + pbuf[pl.ds(5 * SEG_W + r, L)])
            a2 = (pbuf[pl.ds(2 * SEG_W + r, L)]
                  + pbuf[pl.ds(6 * SEG_W + r, L)])
            cnt = (pbuf[pl.ds(3 * SEG_W + r, L)]
                   + pbuf[pl.ds(7 * SEG_W + r, L)])
            rec = 1.0 / jnp.maximum(cnt, 1.0)
            obuf[pl.ds(r, L)] = a0 * rec
            obuf[pl.ds(SEG_W + r, L)] = a1 * rec
            obuf[pl.ds(2 * SEG_W + r, L)] = a2 * rec
            return carry

        lax.fori_loop(0, nseg // L, grp, 0)
        for cc in range(3):
            pltpu.sync_copy(
                obuf.at[pl.ds(cc * SEG_W, nseg)],
                out_hbm.at[pl.ds(cc * NSEG + seg0, nseg)])

    @pl.when(w < NW - 1)
    def _():
        run(SEG_W)

    @pl.when(w == NW - 1)
    def _():
        run(SEG_LAST)


def _run_accum(coor_flat, edges_flat, wvec, zeros):
    accum = pl.kernel(
        _accum_body,
        out_type=jax.ShapeDtypeStruct((NC * SEG_PAD * 4,), jnp.float32),
        mesh=plsc.VectorSubcoreMesh(core_axis_name="c",
                                    subcore_axis_name="s"),
        compiler_params=pltpu.CompilerParams(needs_layout_passes=False),
        scratch_types=[
            pltpu.VMEM_SHARED((SEG_PAD,), jnp.float32),
            pltpu.VMEM_SHARED((SEG_PAD,), jnp.float32),
            pltpu.VMEM_SHARED((SEG_PAD,), jnp.float32),
            pltpu.VMEM_SHARED((SEG_PAD,), jnp.float32),
            pltpu.VMEM((T * 3,), jnp.float32),
            pltpu.VMEM((T,), jnp.int32),
            pltpu.VMEM((T * 4,), jnp.float32),
            pltpu.VMEM((K, B), jnp.int32),
            pltpu.VMEM((T * 3,), jnp.float32),
            pltpu.VMEM((T,), jnp.int32),
            pltpu.VMEM((T * 4,), jnp.float32),
            pltpu.VMEM((K, B), jnp.int32),
            pltpu.VMEM((12, L), jnp.float32),
            pltpu.SemaphoreType.DMA,
            pltpu.SemaphoreType.DMA,
            pltpu.SemaphoreType.DMA,
        ],
    )
    return accum(coor_flat, edges_flat, wvec, zeros)


def _run_final(part):
    final = pl.kernel(
        _final_body,
        out_type=jax.ShapeDtypeStruct((3 * NSEG,), jnp.float32),
        mesh=plsc.VectorSubcoreMesh(core_axis_name="c",
                                    subcore_axis_name="s"),
        compiler_params=pltpu.CompilerParams(needs_layout_passes=False),
        scratch_types=[
            pltpu.VMEM((8 * SEG_W,), jnp.float32),
            pltpu.VMEM((3 * SEG_W,), jnp.float32),
            pltpu.SemaphoreType.DMA,
        ],
    )
    return final(part)


@jax.jit
def kernel(edges, coor, W1, b1, W2, b2, W4, b4):
    Wc = W4 @ W2 @ W1
    bc = W4 @ (W2 @ b1 + b2) + b4
    wvec = jnp.broadcast_to(
        jnp.concatenate([Wc.reshape(9), bc])[:, None], (12, L))
    coor_flat = jnp.transpose(coor, (0, 2, 1)).reshape(3 * E_TOTAL)
    edges_flat = edges.reshape(E_TOTAL)
    zeros = jnp.zeros((SEG_PAD,), jnp.float32)
    part = _run_accum(coor_flat, edges_flat, wvec, zeros)
    out3 = _run_final(part)
    return jnp.transpose(out3.reshape(3, NSEG))[None]


# speedup vs baseline: 46.8243x; 1.3904x over previous
import jax
import jax.numpy as jnp
from jax import lax
from jax.experimental import pallas as pl
from jax.experimental.pallas import tpu as pltpu
from jax.experimental.pallas import tpu_sc as plsc

NSEG = 100000
E_TOTAL = 6400000
NC, NS, L = 2, 16, 16
NW = NC * NS
B = 128
K = 16
T = K * B
UNITS = E_TOTAL // B
BASE_UNITS = UNITS // NW
EXTRA = UNITS - BASE_UNITS * NW
FULL_TILES = BASE_UNITS // K
SEG_PAD = 100352
STRIPE4 = SEG_PAD * 4 // NS
SEG_W = 3136
SEG_LAST = NSEG - 31 * SEG_W


def _accum_body(coor_hbm, edges_hbm, wvec_hbm, zeros_hbm, part_hbm,
                acc0, acc1, acc2, acc3,
                cbuf0, ibuf0, vbuf0, wibuf0,
                cbuf1, ibuf1, vbuf1, wibuf1,
                wbuf, sem, sem0, sem1):
    c = lax.axis_index("c")
    s = lax.axis_index("s")
    w = s * NC + c
    accs = (acc0, acc1, acc2, acc3)

    stripe = SEG_PAD // NS
    for a in accs:
        pltpu.sync_copy(zeros_hbm.at[pl.ds(s * stripe, stripe)],
                        a.at[pl.ds(s * stripe, stripe)])
    pltpu.sync_copy(wvec_hbm, wbuf)
    plsc.subcore_barrier()

    w00 = wbuf[0, :]
    w01 = wbuf[1, :]
    w02 = wbuf[2, :]
    w10 = wbuf[3, :]
    w11 = wbuf[4, :]
    w12 = wbuf[5, :]
    w20 = wbuf[6, :]
    w21 = wbuf[7, :]
    w22 = wbuf[8, :]
    wb0 = wbuf[9, :]
    wb1 = wbuf[10, :]
    wb2 = wbuf[11, :]
    fones = w00 * 0.0 + 1.0

    def _ones(r, carry):
        vbuf0[pl.ds(3 * T + r * L, L)] = fones
        vbuf1[pl.ds(3 * T + r * L, L)] = fones
        return carry

    lax.fori_loop(0, T // L, _ones, 0)

    unit0 = w * BASE_UNITS + jnp.minimum(w, EXTRA)
    n_units = BASE_UNITS + (w < EXTRA).astype(jnp.int32)

    def load_tile(start_unit, k, cbuf, ibuf):
        row0 = start_unit * B
        descs = [
            pltpu.async_copy(coor_hbm.at[pl.ds(row0, k * B)],
                             cbuf.at[pl.ds(0, k * B)], sem),
            pltpu.async_copy(coor_hbm.at[pl.ds(E_TOTAL + row0, k * B)],
                             cbuf.at[pl.ds(T, k * B)], sem),
            pltpu.async_copy(coor_hbm.at[pl.ds(2 * E_TOTAL + row0, k * B)],
                             cbuf.at[pl.ds(2 * T, k * B)], sem),
            pltpu.async_copy(edges_hbm.at[pl.ds(row0, k * B)],
                             ibuf.at[pl.ds(0, k * B)], sem),
        ]
        for d in descs:
            d.wait()

    def compute_tile(k, cbuf, ibuf, vbuf, wibuf):
        def grp(m0, carry):
            for u in range(4):
                m = m0 * 4 + u
                r = m * L
                x0 = cbuf[pl.ds(r, L)]
                x1 = cbuf[pl.ds(T + r, L)]
                x2 = cbuf[pl.ds(2 * T + r, L)]
                vbuf[pl.ds(r, L)] = x0 * w00 + x1 * w01 + x2 * w02 + wb0
                vbuf[pl.ds(T + r, L)] = x0 * w10 + x1 * w11 + x2 * w12 + wb1
                vbuf[pl.ds(2 * T + r, L)] = (
                    x0 * w20 + x1 * w21 + x2 * w22 + wb2)
            return carry

        lax.fori_loop(0, k * 2, grp, 0)

    def stream_tile(k, vbuf, ibuf, wibuf, ssem, start):
        if k == K:
            for cc in range(4):
                d = pltpu.make_async_copy(
                    vbuf.at[pl.ds(cc * T, T)], accs[cc].at[ibuf], ssem)
                if start:
                    d.start(add=True)
                else:
                    d.wait()
        else:
            for cc in range(4):
                d = pltpu.make_async_copy(
                    vbuf.at[pl.ds(cc * T, k * B)],
                    accs[cc].at[wibuf.at[0]], ssem)
                if start:
                    d.start(add=True)
                else:
                    d.wait()

    bufs = ((cbuf0, ibuf0, vbuf0, wibuf0, sem0),
            (cbuf1, ibuf1, vbuf1, wibuf1, sem1))

    def pair(t, carry):
        for p in range(2):
            cbuf, ibuf, vbuf, wibuf, ssem = bufs[p]

            @pl.when(t > 0)
            def _():
                stream_tile(K, vbuf, ibuf, wibuf, ssem, start=False)

            load_tile(unit0 + (t * 2 + p) * K, K, cbuf, ibuf)
            compute_tile(K, cbuf, ibuf, vbuf, wibuf)
            stream_tile(K, vbuf, ibuf, wibuf, ssem, start=True)
        return carry

    npairs = FULL_TILES // 2
    lax.fori_loop(0, npairs, pair, 0)
    stream_tile(K, vbuf0, ibuf0, wibuf0, sem0, start=False)
    stream_tile(K, vbuf1, ibuf1, wibuf1, sem1, start=False)

    def tail_tile(start_unit, k):
        load_tile(start_unit, k, cbuf0, ibuf0)
        compute_tile(k, cbuf0, ibuf0, vbuf0, wibuf0)

        def cpids(g, carry):
            wibuf0[0, pl.ds(g * L, L)] = ibuf0[pl.ds(g * L, L)]
            return carry

        lax.fori_loop(0, (k * B) // L, cpids, 0)
        stream_tile(k, vbuf0, ibuf0, wibuf0, sem0, start=True)
        stream_tile(k, vbuf0, ibuf0, wibuf0, sem0, start=False)

    tail_tile(unit0 + npairs * 2 * K, K)

    def rem_unit(u, carry):
        tail_tile(unit0 + FULL_TILES * K + u, 1)
        return carry

    lax.fori_loop(0, n_units - FULL_TILES * K, rem_unit, 0)

    plsc.subcore_barrier()
    for cc in range(4):
        pltpu.sync_copy(
            accs[cc].at[pl.ds(s * stripe, stripe)],
            part_hbm.at[pl.ds(c * (SEG_PAD * 4) + cc * SEG_PAD + s * stripe,
                              stripe)])


def _final_body(part_hbm, out_hbm, pbuf, obuf, sem):
    c = lax.axis_index("c")
    s = lax.axis_index("s")
    w = s * NC + c
    seg0 = w * SEG_W

    def run(nseg):
        descs = []
        for sc in range(NC):
            for cc in range(4):
                descs.append(pltpu.async_copy(
                    part_hbm.at[pl.ds(
                        sc * (SEG_PAD * 4) + cc * SEG_PAD + seg0, nseg)],
                    pbuf.at[pl.ds((sc * 4 + cc) * SEG_W, nseg)], sem))
        for d in descs:
            d.wait()

        def grp(g, carry):
            r = g * L
            a0 = pbuf[pl.ds(r, L)] + pbuf[pl.ds(4 * SEG_W + r, L)]
            a1 = (pbuf[pl.ds(SEG_W + r, L)]
                  + pbuf[pl.ds(5 * SEG_W + r, L)])
            a2 = (pbuf[pl.ds(2 * SEG_W + r, L)]
                  + pbuf[pl.ds(6 * SEG_W + r, L)])
            cnt = (pbuf[pl.ds(3 * SEG_W + r, L)]
                   + pbuf[pl.ds(7 * SEG_W + r, L)])
            rec = 1.0 / jnp.maximum(cnt, 1.0)
            obuf[pl.ds(r, L)] = a0 * rec
            obuf[pl.ds(SEG_W + r, L)] = a1 * rec
            obuf[pl.ds(2 * SEG_W + r, L)] = a2 * rec
            return carry

        lax.fori_loop(0, nseg // L, grp, 0)
        for cc in range(3):
            pltpu.sync_copy(
                obuf.at[pl.ds(cc * SEG_W, nseg)],
                out_hbm.at[pl.ds(cc * NSEG + seg0, nseg)])

    @pl.when(w < NW - 1)
    def _():
        run(SEG_W)

    @pl.when(w == NW - 1)
    def _():
        run(SEG_LAST)


def _run_accum(coor_flat, edges_flat, wvec, zeros):
    accum = pl.kernel(
        _accum_body,
        out_type=jax.ShapeDtypeStruct((NC * SEG_PAD * 4,), jnp.float32),
        mesh=plsc.VectorSubcoreMesh(core_axis_name="c",
                                    subcore_axis_name="s"),
        compiler_params=pltpu.CompilerParams(needs_layout_passes=False),
        scratch_types=[
            pltpu.VMEM_SHARED((SEG_PAD,), jnp.float32),
            pltpu.VMEM_SHARED((SEG_PAD,), jnp.float32),
            pltpu.VMEM_SHARED((SEG_PAD,), jnp.float32),
            pltpu.VMEM_SHARED((SEG_PAD,), jnp.float32),
            pltpu.VMEM((T * 3,), jnp.float32),
            pltpu.VMEM((T,), jnp.int32),
            pltpu.VMEM((T * 4,), jnp.float32),
            pltpu.VMEM((K, B), jnp.int32),
            pltpu.VMEM((T * 3,), jnp.float32),
            pltpu.VMEM((T,), jnp.int32),
            pltpu.VMEM((T * 4,), jnp.float32),
            pltpu.VMEM((K, B), jnp.int32),
            pltpu.VMEM((12, L), jnp.float32),
            pltpu.SemaphoreType.DMA,
            pltpu.SemaphoreType.DMA,
            pltpu.SemaphoreType.DMA,
        ],
    )
    return accum(coor_flat, edges_flat, wvec, zeros)


def _run_final(part):
    final = pl.kernel(
        _final_body,
        out_type=jax.ShapeDtypeStruct((3 * NSEG,), jnp.float32),
        mesh=plsc.VectorSubcoreMesh(core_axis_name="c",
                                    subcore_axis_name="s"),
        compiler_params=pltpu.CompilerParams(needs_layout_passes=False),
        scratch_types=[
            pltpu.VMEM((8 * SEG_W,), jnp.float32),
            pltpu.VMEM((3 * SEG_W,), jnp.float32),
            pltpu.SemaphoreType.DMA,
        ],
    )
    return final(part)


@jax.jit
def kernel(edges, coor, W1, b1, W2, b2, W4, b4):
    Wc = W4 @ W2 @ W1
    bc = W4 @ (W2 @ b1 + b2) + b4
    wvec = jnp.broadcast_to(
        jnp.concatenate([Wc.reshape(9), bc])[:, None], (12, L))
    coor_flat = jnp.transpose(coor, (0, 2, 1)).reshape(3 * E_TOTAL)
    edges_flat = edges.reshape(E_TOTAL)
    zeros = jnp.zeros((SEG_PAD,), jnp.float32)
    part = _run_accum(coor_flat, edges_flat, wvec, zeros)
    out3 = _run_final(part)
    return jnp.transpose(out3.reshape(3, NSEG))[None]


# speedup vs baseline: 48.5985x; 1.0379x over previous
import jax
import jax.numpy as jnp
from jax import lax
from jax.experimental import pallas as pl
from jax.experimental.pallas import tpu as pltpu
from jax.experimental.pallas import tpu_sc as plsc

NSEG = 100000
E_TOTAL = 6400000
NC, NS, L = 2, 16, 16
NW = NC * NS
B = 128
K = 16
T = K * B
UNITS = E_TOTAL // B
BASE_UNITS = UNITS // NW
EXTRA = UNITS - BASE_UNITS * NW
FULL_TILES = BASE_UNITS // K
SEG_PAD = 100352
STRIPE4 = SEG_PAD * 4 // NS
SEG_W = 3136
SEG_LAST = NSEG - 31 * SEG_W


def _accum_body(coor_hbm, edges_hbm, wvec_hbm, zeros_hbm, part_hbm,
                acc0, acc1, acc2, acc3,
                cbuf0, ibuf0, vbuf0, wibuf0,
                cbuf1, ibuf1, vbuf1, wibuf1,
                tibuf, wbuf, sem, sem0, sem1, semi0, semi1):
    c = lax.axis_index("c")
    s = lax.axis_index("s")
    w = s * NC + c
    accs = (acc0, acc1, acc2, acc3)

    stripe = SEG_PAD // NS
    for a in accs:
        pltpu.sync_copy(zeros_hbm.at[pl.ds(s * stripe, stripe)],
                        a.at[pl.ds(s * stripe, stripe)])
    pltpu.sync_copy(wvec_hbm, wbuf)
    plsc.subcore_barrier()

    w00 = wbuf[0, :]
    w01 = wbuf[1, :]
    w02 = wbuf[2, :]
    w10 = wbuf[3, :]
    w11 = wbuf[4, :]
    w12 = wbuf[5, :]
    w20 = wbuf[6, :]
    w21 = wbuf[7, :]
    w22 = wbuf[8, :]
    wb0 = wbuf[9, :]
    wb1 = wbuf[10, :]
    wb2 = wbuf[11, :]
    fones = w00 * 0.0 + 1.0

    def _ones(r, carry):
        vbuf0[pl.ds(3 * T + r * L, L)] = fones
        vbuf1[pl.ds(3 * T + r * L, L)] = fones
        return carry

    lax.fori_loop(0, T // L, _ones, 0)

    unit0 = w * BASE_UNITS + jnp.minimum(w, EXTRA)
    n_units = BASE_UNITS + (w < EXTRA).astype(jnp.int32)

    def load_descs(start_unit, k, cbuf, ibuf, insem):
        row0 = start_unit * B
        return [
            pltpu.make_async_copy(coor_hbm.at[pl.ds(row0, k * B)],
                                  cbuf.at[pl.ds(0, k * B)], insem),
            pltpu.make_async_copy(coor_hbm.at[pl.ds(E_TOTAL + row0, k * B)],
                                  cbuf.at[pl.ds(T, k * B)], insem),
            pltpu.make_async_copy(
                coor_hbm.at[pl.ds(2 * E_TOTAL + row0, k * B)],
                cbuf.at[pl.ds(2 * T, k * B)], insem),
            pltpu.make_async_copy(edges_hbm.at[pl.ds(row0, k * B)],
                                  ibuf.at[pl.ds(0, k * B)], insem),
        ]

    def load_issue(start_unit, k, cbuf, ibuf, insem):
        start_unit = jnp.minimum(start_unit, UNITS - k)
        for d in load_descs(start_unit, k, cbuf, ibuf, insem):
            d.start()

    def load_wait(k, cbuf, ibuf, insem):
        for d in load_descs(0, k, cbuf, ibuf, insem):
            d.wait()

    def load_tile(start_unit, k, cbuf, ibuf):
        load_issue(start_unit, k, cbuf, ibuf, sem)
        load_wait(k, cbuf, ibuf, sem)

    def compute_tile(k, cbuf, ibuf, vbuf, wibuf):
        def grp(m0, carry):
            for u in range(4):
                m = m0 * 4 + u
                r = m * L
                x0 = cbuf[pl.ds(r, L)]
                x1 = cbuf[pl.ds(T + r, L)]
                x2 = cbuf[pl.ds(2 * T + r, L)]
                vbuf[pl.ds(r, L)] = x0 * w00 + x1 * w01 + x2 * w02 + wb0
                vbuf[pl.ds(T + r, L)] = x0 * w10 + x1 * w11 + x2 * w12 + wb1
                vbuf[pl.ds(2 * T + r, L)] = (
                    x0 * w20 + x1 * w21 + x2 * w22 + wb2)
                wibuf[pl.ds(r, L)] = ibuf[pl.ds(r, L)]
            return carry

        lax.fori_loop(0, k * 2, grp, 0)

    def stream_tile(k, vbuf, wibuf, ssem, start):
        for cc in range(4):
            d = pltpu.make_async_copy(
                vbuf.at[pl.ds(cc * T, k * B)], accs[cc].at[wibuf], ssem)
            if start:
                d.start(add=True)
            else:
                d.wait()

    bufs = ((cbuf0, ibuf0, vbuf0, wibuf0, sem0, semi0),
            (cbuf1, ibuf1, vbuf1, wibuf1, sem1, semi1))

    load_issue(unit0, K, cbuf0, ibuf0, semi0)
    load_issue(unit0 + K, K, cbuf1, ibuf1, semi1)

    def pair(t, carry):
        for p in range(2):
            cbuf, ibuf, vbuf, wibuf, ssem, isem = bufs[p]

            @pl.when(t > 0)
            def _():
                stream_tile(K, vbuf, wibuf, ssem, start=False)

            load_wait(K, cbuf, ibuf, isem)
            compute_tile(K, cbuf, ibuf, vbuf, wibuf)
            stream_tile(K, vbuf, wibuf, ssem, start=True)
            load_issue(unit0 + (t * 2 + p + 2) * K, K, cbuf, ibuf, isem)
        return carry

    npairs = FULL_TILES // 2
    lax.fori_loop(0, npairs, pair, 0)
    load_wait(K, cbuf0, ibuf0, semi0)
    load_wait(K, cbuf1, ibuf1, semi1)
    stream_tile(K, vbuf0, wibuf0, sem0, start=False)
    stream_tile(K, vbuf1, wibuf1, sem1, start=False)

    def tail_tile(start_unit, k):
        load_tile(start_unit, k, cbuf0, ibuf0)
        compute_tile(k, cbuf0, ibuf0, vbuf0, wibuf0)
        if k == K:
            idref = wibuf0
        else:
            idref = tibuf

            def cpids(g, carry):
                tibuf[pl.ds(g * L, L)] = ibuf0[pl.ds(g * L, L)]
                return carry

            lax.fori_loop(0, (k * B) // L, cpids, 0)
        stream_tile(k, vbuf0, idref, sem0, start=True)
        stream_tile(k, vbuf0, idref, sem0, start=False)

    tail_tile(unit0 + npairs * 2 * K, K)

    def rem_unit(u, carry):
        tail_tile(unit0 + FULL_TILES * K + u, 1)
        return carry

    lax.fori_loop(0, n_units - FULL_TILES * K, rem_unit, 0)

    plsc.subcore_barrier()
    for cc in range(4):
        pltpu.sync_copy(
            accs[cc].at[pl.ds(s * stripe, stripe)],
            part_hbm.at[pl.ds(c * (SEG_PAD * 4) + cc * SEG_PAD + s * stripe,
                              stripe)])


def _final_body(part_hbm, out_hbm, pbuf, obuf, sem):
    c = lax.axis_index("c")
    s = lax.axis_index("s")
    w = s * NC + c
    seg0 = w * SEG_W

    def run(nseg):
        descs = []
        for sc in range(NC):
            for cc in range(4):
                descs.append(pltpu.async_copy(
                    part_hbm.at[pl.ds(
                        sc * (SEG_PAD * 4) + cc * SEG_PAD + seg0, nseg)],
                    pbuf.at[pl.ds((sc * 4 + cc) * SEG_W, nseg)], sem))
        for d in descs:
            d.wait()

        def grp(g, carry):
            r = g * L
            a0 = pbuf[pl.ds(r, L)] + pbuf[pl.ds(4 * SEG_W + r, L)]
            a1 = (pbuf[pl.ds(SEG_W + r, L)]
                  + pbuf[pl.ds(5 * SEG_W + r, L)])
            a2 = (pbuf[pl.ds(2 * SEG_W + r, L)]
                  + pbuf[pl.ds(6 * SEG_W + r, L)])
            cnt = (pbuf[pl.ds(3 * SEG_W + r, L)]
                   + pbuf[pl.ds(7 * SEG_W + r, L)])
            rec = 1.0 / jnp.maximum(cnt, 1.0)
            obuf[pl.ds(r, L)] = a0 * rec
            obuf[pl.ds(SEG_W + r, L)] = a1 * rec
            obuf[pl.ds(2 * SEG_W + r, L)] = a2 * rec
            return carry

        lax.fori_loop(0, nseg // L, grp, 0)
        for cc in range(3):
            pltpu.sync_copy(
                obuf.at[pl.ds(cc * SEG_W, nseg)],
                out_hbm.at[pl.ds(cc * NSEG + seg0, nseg)])

    @pl.when(w < NW - 1)
    def _():
        run(SEG_W)

    @pl.when(w == NW - 1)
    def _():
        run(SEG_LAST)


def _run_accum(coor_flat, edges_flat, wvec, zeros):
    accum = pl.kernel(
        _accum_body,
        out_type=jax.ShapeDtypeStruct((NC * SEG_PAD * 4,), jnp.float32),
        mesh=plsc.VectorSubcoreMesh(core_axis_name="c",
                                    subcore_axis_name="s"),
        compiler_params=pltpu.CompilerParams(needs_layout_passes=False),
        scratch_types=[
            pltpu.VMEM_SHARED((SEG_PAD,), jnp.float32),
            pltpu.VMEM_SHARED((SEG_PAD,), jnp.float32),
            pltpu.VMEM_SHARED((SEG_PAD,), jnp.float32),
            pltpu.VMEM_SHARED((SEG_PAD,), jnp.float32),
            pltpu.VMEM((T * 3,), jnp.float32),
            pltpu.VMEM((T,), jnp.int32),
            pltpu.VMEM((T * 4,), jnp.float32),
            pltpu.VMEM((T,), jnp.int32),
            pltpu.VMEM((T * 3,), jnp.float32),
            pltpu.VMEM((T,), jnp.int32),
            pltpu.VMEM((T * 4,), jnp.float32),
            pltpu.VMEM((T,), jnp.int32),
            pltpu.VMEM((B,), jnp.int32),
            pltpu.VMEM((12, L), jnp.float32),
            pltpu.SemaphoreType.DMA,
            pltpu.SemaphoreType.DMA,
            pltpu.SemaphoreType.DMA,
            pltpu.SemaphoreType.DMA,
            pltpu.SemaphoreType.DMA,
        ],
    )
    return accum(coor_flat, edges_flat, wvec, zeros)


def _run_final(part):
    final = pl.kernel(
        _final_body,
        out_type=jax.ShapeDtypeStruct((3 * NSEG,), jnp.float32),
        mesh=plsc.VectorSubcoreMesh(core_axis_name="c",
                                    subcore_axis_name="s"),
        compiler_params=pltpu.CompilerParams(needs_layout_passes=False),
        scratch_types=[
            pltpu.VMEM((8 * SEG_W,), jnp.float32),
            pltpu.VMEM((3 * SEG_W,), jnp.float32),
            pltpu.SemaphoreType.DMA,
        ],
    )
    return final(part)


@jax.jit
def kernel(edges, coor, W1, b1, W2, b2, W4, b4):
    Wc = W4 @ W2 @ W1
    bc = W4 @ (W2 @ b1 + b2) + b4
    wvec = jnp.broadcast_to(
        jnp.concatenate([Wc.reshape(9), bc])[:, None], (12, L))
    coor_flat = jnp.transpose(coor, (0, 2, 1)).reshape(3 * E_TOTAL)
    edges_flat = edges.reshape(E_TOTAL)
    zeros = jnp.zeros((SEG_PAD,), jnp.float32)
    part = _run_accum(coor_flat, edges_flat, wvec, zeros)
    out3 = _run_final(part)
    return jnp.transpose(out3.reshape(3, NSEG))[None]
